# baseline (speedup 1.0000x reference)
import jax
import jax.numpy as jnp
from jax import lax
from jax.experimental import pallas as pl
from jax.experimental.pallas import tpu as pltpu

H = 32
OUT = 64
T = 8


def _lstm_body(seq_ref,
               chunks_ref,
               wih0_ref,
               whh0_ref,
               b0_ref,
               w1_ref,
               b1_ref,
               wout_ref,
               bout_ref,
               out_ref,
               wscr):
    f32 = jnp.float32

    iota_sub = lax.broadcasted_iota(jnp.int32, (8, H), 0)
    rows = []
    for t in range(T):
        sub = seq_ref[t] & 7
        mask = (iota_sub == sub).astype(f32)
        rows.append(jnp.sum(chunks_ref[t * 8:(t + 1) * 8, :] * mask,
                            axis=0, keepdims=True))
    x = jnp.concatenate(rows, axis=0)

    def gate_cols(m):
        return [m[:, k * H:(k + 1) * H] for k in range(4)]

    g0 = gate_cols(jnp.dot(x, wih0_ref[...], preferred_element_type=f32)
                   + b0_ref[...])
    b1_k = gate_cols(b1_ref[...])

    whh0_k = gate_cols(whh0_ref[...])
    wih1_k = gate_cols(w1_ref[:H, :])
    whh1_k = gate_cols(w1_ref[H:, :])
    for k in range(4):
        wscr[(0 + k) * H:(1 + k) * H, :] = whh0_k[k]
        wscr[(4 + k) * H:(5 + k) * H, :] = wih1_k[k]
        wscr[(8 + k) * H:(9 + k) * H, :] = whh1_k[k]

    def combine(pre, c):
        i = jax.nn.sigmoid(pre[0])
        f = jax.nn.sigmoid(pre[1])
        g = jnp.tanh(pre[2])
        o = jax.nn.sigmoid(pre[3])
        c_new = f * c + i * g
        return o * jnp.tanh(c_new), c_new

    def mm(a, b):
        return jnp.dot(a, b, preferred_element_type=f32)

    def wmat(j):
        return wscr[j * H:(j + 1) * H, :]

    def rep8(v):
        return jnp.broadcast_to(v, (8, H))

    zero = jnp.zeros((8, H), f32)
    h0, c0, h1, c1 = zero, zero, zero, zero
    hs1 = []
    for t in range(T):
        pre0 = [rep8(g0[k][t:t + 1, :]) + mm(h0, wmat(k)) for k in range(4)]
        h0, c0 = combine(pre0, c0)
        pre1 = [rep8(b1_k[k]) + mm(h0, wmat(4 + k)) + mm(h1, wmat(8 + k))
                for k in range(4)]
        h1, c1 = combine(pre1, c1)
        hs1.append(h1[0:1, :])
    h1_all = jnp.concatenate(hs1, axis=0)

    res = (jnp.dot(h1_all, wout_ref[...], preferred_element_type=f32)
           + bout_ref[...])
    out_ref[...] = res[:, :OUT]


def kernel(seq, embedding, wih0_t, whh0_t, b0, w1_fused, b1, wout_pad_t,
           bout_pad):
    bases = (seq >> 3) << 3
    chunks = jax.vmap(
        lambda b: lax.dynamic_slice(embedding, (b, 0), (8, H))
    )(bases).reshape(T * 8, H)

    vmem_full = lambda shape: pl.BlockSpec(shape,
                                           lambda i, s: tuple(0 for _ in shape))

    grid_spec = pltpu.PrefetchScalarGridSpec(
        num_scalar_prefetch=1,
        grid=(1,),
        in_specs=[
            vmem_full((T * 8, H)),
            vmem_full((H, 4 * H)),
            vmem_full((H, 4 * H)),
            vmem_full((1, 4 * H)),
            vmem_full((2 * H, 4 * H)),
            vmem_full((1, 4 * H)),
            vmem_full((H, 4 * H)),
            vmem_full((1, 4 * H)),
        ],
        out_specs=vmem_full((T, OUT)),
        scratch_shapes=[
            pltpu.VMEM((12 * H, H), jnp.float32),
        ],
    )

    out = pl.pallas_call(
        _lstm_body,
        out_shape=jax.ShapeDtypeStruct((T, OUT), jnp.float32),
        grid_spec=grid_spec,
        compiler_params=pltpu.CompilerParams(
            dimension_semantics=("arbitrary",)),
    )(seq, chunks, wih0_t, whh0_t, b0, w1_fused, b1, wout_pad_t, bout_pad)
    return out


# speedup vs baseline: 1.9244x; 1.9244x over previous
import jax
import jax.numpy as jnp
from jax import lax
from jax.experimental import pallas as pl
from jax.experimental.pallas import tpu as pltpu

H = 32
OUT = 64
T = 8


def _lstm_body(seq_ref,
               chunks_ref,
               wih0_ref,
               whh0_ref,
               b0_ref,
               w1_ref,
               b1_ref,
               wout_ref,
               bout_ref,
               out_ref,
               wscr):
    f32 = jnp.float32

    iota_sub = lax.broadcasted_iota(jnp.int32, (8, H), 0)
    rows = []
    for t in range(T):
        sub = seq_ref[t] & 7
        mask = (iota_sub == sub).astype(f32)
        rows.append(jnp.sum(chunks_ref[t * 8:(t + 1) * 8, :] * mask,
                            axis=0, keepdims=True))
    x = jnp.concatenate(rows, axis=0)

    W = 4 * H

    def wide(m, k):
        return jnp.concatenate([m[:, k * H:(k + 1) * H]] * 4, axis=1)

    for k in range(4):
        wscr[(0 + k) * H:(1 + k) * H, :] = wide(wih0_ref[...], k)
        wscr[(4 + k) * H:(5 + k) * H, :] = wide(whh0_ref[...], k)
        wscr[(8 + k) * H:(9 + k) * H, :] = wide(w1_ref[:H, :], k)
        wscr[(12 + k) * H:(13 + k) * H, :] = wide(w1_ref[H:, :], k)

    def bias_wide(b, k):
        return jnp.concatenate([b[:, k * H:(k + 1) * H]] * 4, axis=1)

    b0_k = [bias_wide(b0_ref[...], k) for k in range(4)]
    b1_k = [bias_wide(b1_ref[...], k) for k in range(4)]

    def combine(pre, c):
        i = jax.nn.sigmoid(pre[0])
        f = jax.nn.sigmoid(pre[1])
        g = jnp.tanh(pre[2])
        o = jax.nn.sigmoid(pre[3])
        c_new = f * c + i * g
        return o * jnp.tanh(c_new), c_new

    def mm(a, b):
        return jnp.dot(a, b, preferred_element_type=f32)

    def wmat(j):
        return wscr[j * H:(j + 1) * H, :]

    def rep8(v):
        return jnp.broadcast_to(v, (8, W))

    g0 = [jnp.dot(x, wmat(k), preferred_element_type=f32) + b0_k[k]
          for k in range(4)]

    zero = jnp.zeros((8, W), f32)
    h0, c0, h1, c1 = zero, zero, zero, zero
    hs1 = []
    for t in range(T):
        hn0 = h0[:, :H]
        pre0 = [rep8(g0[k][t:t + 1, :]) + mm(hn0, wmat(4 + k))
                for k in range(4)]
        h0, c0 = combine(pre0, c0)
        hn0 = h0[:, :H]
        hn1 = h1[:, :H]
        pre1 = [rep8(b1_k[k]) + mm(hn0, wmat(8 + k)) + mm(hn1, wmat(12 + k))
                for k in range(4)]
        h1, c1 = combine(pre1, c1)
        hs1.append(h1[0:1, :H])
    h1_all = jnp.concatenate(hs1, axis=0)

    res = (jnp.dot(h1_all, wout_ref[...], preferred_element_type=f32)
           + bout_ref[...])
    out_ref[...] = res[:, :OUT]


def kernel(seq, embedding, wih0_t, whh0_t, b0, w1_fused, b1, wout_pad_t,
           bout_pad):
    chunk_list = [
        lax.dynamic_slice_in_dim(embedding, (seq[t] >> 3) << 3, 8, axis=0)
        for t in range(T)
    ]
    chunks = jnp.concatenate(chunk_list, axis=0)

    vmem_full = lambda shape: pl.BlockSpec(shape,
                                           lambda i, s: tuple(0 for _ in shape))

    grid_spec = pltpu.PrefetchScalarGridSpec(
        num_scalar_prefetch=1,
        grid=(1,),
        in_specs=[
            vmem_full((T * 8, H)),
            vmem_full((H, 4 * H)),
            vmem_full((H, 4 * H)),
            vmem_full((1, 4 * H)),
            vmem_full((2 * H, 4 * H)),
            vmem_full((1, 4 * H)),
            vmem_full((H, 4 * H)),
            vmem_full((1, 4 * H)),
        ],
        out_specs=vmem_full((T, OUT)),
        scratch_shapes=[
            pltpu.VMEM((16 * H, 4 * H), jnp.float32),
        ],
    )

    out = pl.pallas_call(
        _lstm_body,
        out_shape=jax.ShapeDtypeStruct((T, OUT), jnp.float32),
        grid_spec=grid_spec,
        compiler_params=pltpu.CompilerParams(
            dimension_semantics=("arbitrary",)),
    )(seq, chunks, wih0_t, whh0_t, b0, w1_fused, b1, wout_pad_t, bout_pad)
    return out
